# baseline (speedup 1.0000x reference)
import functools

import jax
import jax.numpy as jnp
from jax import lax
from jax.experimental import pallas as pl
from jax.experimental.pallas import tpu as pltpu
from jax.experimental.pallas import tpu_sc as plsc


_NC = 2
_NS = 16
_NW = _NC * _NS
_CH = 2000


def kernel(x):
    n, d = x.shape
    nch = n // _CH
    mesh = plsc.VectorSubcoreMesh(core_axis_name="c", subcore_axis_name="s")

    @functools.partial(
        pl.kernel,
        out_type=jax.ShapeDtypeStruct((n, d), x.dtype),
        mesh=mesh,
        scratch_types=[
            pltpu.VMEM((8, d), x.dtype),
            pltpu.SemaphoreType.DMA,
            pltpu.SemaphoreType.DMA,
        ],
    )
    def _copy(x_hbm, o_hbm, ones_buf, sem, head_sem):
        wid = lax.axis_index("s") * _NC + lax.axis_index("c")
        n_my = (nch - wid + _NW - 1) // _NW

        def chunk_copy(k):
            row = (wid + k * _NW) * _CH
            return pltpu.make_async_copy(
                x_hbm.at[pl.ds(row, _CH), :],
                o_hbm.at[pl.ds(row, _CH), :],
                sem,
            )

        def fire(k, carry):
            chunk_copy(k).start()
            return carry

        lax.fori_loop(0, n_my, fire, 0)

        def drain(k, carry):
            chunk_copy(k).wait()
            return carry

        lax.fori_loop(0, n_my, drain, 0)

        @pl.when(wid == 0)
        def _():
            ones = jnp.ones((16,), x.dtype)
            for r in range(2):
                for j in range(d // 16):
                    ones_buf[r, pl.ds(16 * j, 16)] = ones
            head = pltpu.make_async_copy(
                ones_buf.at[pl.ds(0, 2), :], o_hbm.at[pl.ds(0, 2), :], head_sem
            )
            head.start()
            head.wait()

    return _copy(x)


# speedup vs baseline: 15.3453x; 15.3453x over previous
import functools

import jax
import jax.numpy as jnp
from jax import lax
from jax.experimental import pallas as pl
from jax.experimental.pallas import tpu as pltpu
from jax.experimental.pallas import tpu_sc as plsc


_NC = 2
_NS = 16
_NW = _NC * _NS
_CH = 400
_NBUF = 2


def kernel(x):
    n, d = x.shape
    nch = n // _CH
    mesh = plsc.VectorSubcoreMesh(core_axis_name="c", subcore_axis_name="s")

    @functools.partial(
        pl.kernel,
        out_type=jax.ShapeDtypeStruct((n, d), x.dtype),
        mesh=mesh,
        scratch_types=[
            pltpu.VMEM((_NBUF, _CH, d), x.dtype),
            pltpu.SemaphoreType.DMA((_NBUF,)),
            pltpu.SemaphoreType.DMA((_NBUF,)),
        ],
    )
    def _copy(x_hbm, o_hbm, bufs, in_sems, out_sems):
        wid = lax.axis_index("s") * _NC + lax.axis_index("c")
        n_my = (nch - wid + _NW - 1) // _NW

        def in_start(b, k):
            row = (wid + k * _NW) * _CH
            pltpu.make_async_copy(
                x_hbm.at[pl.ds(row, _CH), :], bufs.at[b], in_sems.at[b]
            ).start()

        for b in range(_NBUF):
            @pl.when(b < n_my)
            def _():
                in_start(b, b)

        def step(k, carry):
            b = lax.rem(k, _NBUF)
            row = (wid + k * _NW) * _CH
            pltpu.make_async_copy(
                x_hbm.at[pl.ds(row, _CH), :], bufs.at[b], in_sems.at[b]
            ).wait()

            @pl.when(jnp.logical_and(wid == 0, k == 0))
            def _():
                ones = jnp.ones((16,), x.dtype)
                for r in range(2):
                    for j in range(d // 16):
                        bufs[0, r, pl.ds(16 * j, 16)] = ones

            out_cp = pltpu.make_async_copy(
                bufs.at[b], o_hbm.at[pl.ds(row, _CH), :], out_sems.at[b]
            )
            out_cp.start()
            out_cp.wait()

            @pl.when(k + _NBUF < n_my)
            def _():
                in_start(b, k + _NBUF)

            return carry

        lax.fori_loop(0, n_my, step, 0)

    return _copy(x)


# speedup vs baseline: 15.3512x; 1.0004x over previous
import functools

import jax
import jax.numpy as jnp
from jax import lax
from jax.experimental import pallas as pl
from jax.experimental.pallas import tpu as pltpu
from jax.experimental.pallas import tpu_sc as plsc


_NC = 2
_NS = 16
_NW = _NC * _NS
_CH = 200
_NBUF = 4


def kernel(x):
    n, d = x.shape
    nch = n // _CH
    mesh = plsc.VectorSubcoreMesh(core_axis_name="c", subcore_axis_name="s")

    @functools.partial(
        pl.kernel,
        out_type=jax.ShapeDtypeStruct((n, d), x.dtype),
        mesh=mesh,
        scratch_types=[
            pltpu.VMEM((_NBUF, _CH, d), x.dtype),
            pltpu.SemaphoreType.DMA((_NBUF,)),
            pltpu.SemaphoreType.DMA((_NBUF,)),
        ],
    )
    def _copy(x_hbm, o_hbm, bufs, in_sems, out_sems):
        wid = lax.axis_index("s") * _NC + lax.axis_index("c")
        n_my = (nch - wid + _NW - 1) // _NW

        def in_start(b, k):
            row = (wid + k * _NW) * _CH
            pltpu.make_async_copy(
                x_hbm.at[pl.ds(row, _CH), :], bufs.at[b], in_sems.at[b]
            ).start()

        for b in range(_NBUF):
            @pl.when(b < n_my)
            def _():
                in_start(b, b)

        def step(k, carry):
            b = lax.rem(k, _NBUF)
            row = (wid + k * _NW) * _CH
            pltpu.make_async_copy(
                x_hbm.at[pl.ds(row, _CH), :], bufs.at[b], in_sems.at[b]
            ).wait()

            @pl.when(jnp.logical_and(wid == 0, k == 0))
            def _():
                ones = jnp.ones((16,), x.dtype)
                for r in range(2):
                    for j in range(d // 16):
                        bufs[0, r, pl.ds(16 * j, 16)] = ones

            pltpu.make_async_copy(
                bufs.at[b], o_hbm.at[pl.ds(row, _CH), :], out_sems.at[b]
            ).start()

            @pl.when(k + _NBUF < n_my)
            def _():
                pltpu.make_async_copy(
                    bufs.at[b], o_hbm.at[pl.ds(row, _CH), :], out_sems.at[b]
                ).wait()
                in_start(b, k + _NBUF)

            return carry

        lax.fori_loop(0, n_my, step, 0)

        for b in range(_NBUF):
            @pl.when(b < n_my)
            def _():
                pltpu.make_async_copy(
                    bufs.at[b], o_hbm.at[pl.ds(0, _CH), :], out_sems.at[b]
                ).wait()

    return _copy(x)


# speedup vs baseline: 16.1627x; 1.0529x over previous
import jax
import jax.numpy as jnp
from jax import lax
from jax.experimental import pallas as pl
from jax.experimental.pallas import tpu as pltpu


_CH = 2000
_K = 8
_D = _K // 2


def kernel(x):
    n, d = x.shape
    nch = n // _CH

    def body(x_ref, o_ref, bufs, in_sems, out_sems):
        def in_cp(k):
            b = lax.rem(k, _K)
            return pltpu.make_async_copy(
                x_ref.at[pl.ds(k * _CH, _CH), :], bufs.at[b], in_sems.at[b]
            )

        def out_cp(k):
            b = lax.rem(k, _K)
            return pltpu.make_async_copy(
                bufs.at[b], o_ref.at[pl.ds(k * _CH, _CH), :], out_sems.at[b]
            )

        for k in range(_D):
            in_cp(k).start()

        def step(k, carry):
            @pl.when(k - _D >= 0)
            def _():
                out_cp(k - _D).wait()

            @pl.when(k + _D < nch)
            def _():
                in_cp(k + _D).start()

            in_cp(k).wait()

            @pl.when(k == 0)
            def _():
                bufs[0, 0:2, :] = jnp.ones((2, d), x.dtype)

            out_cp(k).start()
            return carry

        lax.fori_loop(0, nch, step, 0)

        for j in range(_D):
            out_cp(nch - _D + j).wait()

    return pl.pallas_call(
        body,
        in_specs=[pl.BlockSpec(memory_space=pltpu.MemorySpace.HBM)],
        out_specs=pl.BlockSpec(memory_space=pltpu.MemorySpace.HBM),
        out_shape=jax.ShapeDtypeStruct((n, d), x.dtype),
        scratch_shapes=[
            pltpu.VMEM((_K, _CH, d), x.dtype),
            pltpu.SemaphoreType.DMA((_K,)),
            pltpu.SemaphoreType.DMA((_K,)),
        ],
    )(x)
